# baseline (speedup 1.0000x reference)
import functools

import jax
import jax.numpy as jnp
from jax import lax
from jax.experimental import pallas as pl
from jax.experimental.pallas import tpu as pltpu
from jax.experimental.pallas import tpu_sc as plsc

N = 10000
E = 320000
D_IN = 128
HEADS = 8
C1 = 16
D_OUT = 64

NC = 2
NS = 16
NW = NC * NS

EPW = E // NW
EB1, NCHUNK1 = 80, 126
EB2, NCHUNK2 = 160, 66

ACC_N = 10240
ROWS_PER_TILE = ACC_N // NS
ZBLK = 80

TW1 = 144
TW2 = 80
R = 1000


def _leaky(t):
    return jnp.maximum(t, 0.2 * t)


@functools.lru_cache(maxsize=None)
def _make_edge_kernel(tw, mw, n_heads, EB, NCHUNK):
    nchunks16 = mw // 16

    mesh = plsc.VectorSubcoreMesh(core_axis_name="c", subcore_axis_name="s",
                                  num_cores=NC, num_subcores=NS)

    NGB, NAB, NIB = 3, 2, 6

    scratch = (
        [pltpu.VMEM_SHARED((ACC_N, tw), jnp.float32)]
        + [pltpu.VMEM((2, EB), jnp.int32) for _ in range(NIB)]
        + [pltpu.VMEM((EB, tw), jnp.float32) for _ in range(NGB)]
        + [pltpu.VMEM((EB, 16), jnp.float32) for _ in range(NAB)]
        + [pltpu.SemaphoreType.DMA for _ in range(NIB + NGB + NAB + NGB)]
    )

    @functools.partial(
        pl.kernel,
        out_type=jax.ShapeDtypeStruct((NC, ACC_N, tw), jnp.float32),
        mesh=mesh,
        scratch_types=scratch,
        compiler_params=pltpu.CompilerParams(use_tc_tiling_on_sc=False),
    )
    def edge_kernel(eip_hbm, table_hbm, adt_hbm, acc_hbm, *scr):
        acc_sh = scr[0]
        ibuf = scr[1:1 + NIB]
        g = scr[1 + NIB:1 + NIB + NGB]
        ad = scr[1 + NIB + NGB:1 + NIB + NGB + NAB]
        sems_all = scr[1 + NIB + NGB + NAB:]
        semi = sems_all[0:NIB]
        semg = sems_all[NIB:NIB + NGB]
        sema = sems_all[NIB + NGB:NIB + NGB + NAB]
        sems = sems_all[NIB + NGB + NAB:]

        c = lax.axis_index("c")
        s = lax.axis_index("s")
        wid = c * NS + s

        def idx_copy(ci, q):
            return pltpu.make_async_copy(eip_hbm.at[wid, ci], ibuf[q], semi[q])

        def g_copy(q, pg):
            return pltpu.make_async_copy(table_hbm.at[ibuf[q].at[0]], g[pg],
                                         semg[pg])

        def ad_copy(q, pa):
            return pltpu.make_async_copy(adt_hbm.at[ibuf[q].at[1]], ad[pa],
                                         sema[pa])

        def sc_copy(q, pg):
            return pltpu.make_async_copy(g[pg], acc_sh.at[ibuf[q].at[1]],
                                         sems[pg])

        zero16 = jnp.zeros((16,), jnp.float32)

        def zrow(r, carry):
            for k in range(tw // 16):
                g[0][r, pl.ds(16 * k, 16)] = zero16
            return carry

        lax.fori_loop(0, ZBLK, zrow, 0)
        row0 = s * ROWS_PER_TILE
        for i in range(ROWS_PER_TILE // ZBLK):
            pltpu.sync_copy(g[0].at[pl.ds(0, ZBLK)],
                            acc_sh.at[pl.ds(row0 + i * ZBLK, ZBLK)])
        plsc.subcore_barrier()

        idx_copy(0, 0).start()
        idx_copy(1, 1).start()
        idx_copy(0, 0).wait()
        g_copy(0, 0).start()
        ad_copy(0, 0).start()

        def group(gi, carry):
            for k in range(6):
                i = gi * 6 + k
                pg = k % 3
                pgn = (k + 1) % 3
                pa = k % 2
                pan = (k + 1) % 2
                qi = k
                qn1 = (k + 1) % 6
                qn2 = (k + 2) % 6

                @pl.when(i >= 2)
                def _():
                    sc_copy((k - 2) % 6, pgn).wait()

                @pl.when(i + 1 < NCHUNK)
                def _():
                    idx_copy(i + 1, qn1).wait()
                    g_copy(qn1, pgn).start()
                    ad_copy(qn1, pan).start()

                @pl.when(i + 2 < NCHUNK)
                def _():
                    idx_copy(i + 2, qn2).start()

                g_copy(qi, pg).wait()
                ad_copy(qi, pa).wait()

                gb = g[pg]
                ab = ad[pa]

                def edge_ea(e, ecarry):
                    t = gb[e, pl.ds(mw, 16)] + ab[e, :]
                    gb[e, pl.ds(mw, 16)] = jnp.exp(jnp.maximum(t, 0.2 * t))
                    return ecarry

                def edge_scale(e, ecarry):
                    ea = gb[e, pl.ds(mw, 16)]
                    for kk in range(nchunks16):
                        lane = (kk * n_heads * 16) // mw
                        gb[e, pl.ds(16 * kk, 16)] = (
                            gb[e, pl.ds(16 * kk, 16)] * ea[lane])
                    return ecarry

                lax.fori_loop(0, EB, edge_ea, 0, unroll=8)
                lax.fori_loop(0, EB, edge_scale, 0, unroll=2)
                pltpu.async_copy(gb, acc_sh.at[ibuf[qi].at[1]], sems[pg],
                                 add=True)
            return carry

        lax.fori_loop(0, NCHUNK // 6, group, 0)

        sc_copy((NCHUNK - 2) % 6, (NCHUNK - 2) % 3).wait()
        sc_copy((NCHUNK - 1) % 6, (NCHUNK - 1) % 3).wait()
        plsc.subcore_barrier()

        for i in range(ROWS_PER_TILE // ZBLK):
            r = row0 + i * ZBLK
            pltpu.sync_copy(acc_sh.at[pl.ds(r, ZBLK)],
                            acc_hbm.at[c, pl.ds(r, ZBLK)])

    return edge_kernel


def _head_expand():
    rows = lax.broadcasted_iota(jnp.int32, (HEADS, D_IN), 0)
    cols = lax.broadcasted_iota(jnp.int32, (HEADS, D_IN), 1)
    return (rows == cols // C1).astype(jnp.float32)


def _tc1_body(x_ref, w1_ref, as_ref, ad_ref, t1_ref, adt_ref, es_ref):
    h = jnp.dot(x_ref[...], w1_ref[...], preferred_element_type=jnp.float32)
    a_s = jnp.dot(h, as_ref[...], preferred_element_type=jnp.float32)
    a_d = jnp.dot(h, ad_ref[...], preferred_element_type=jnp.float32)
    zpad = jnp.zeros((R, 8), jnp.float32)
    t1_ref[...] = jnp.concatenate([h, a_s, zpad], axis=1)
    adt_ref[...] = jnp.concatenate([a_d, zpad], axis=1)
    es_ref[...] = jnp.exp(_leaky(a_s + a_d))


def _tc2_body(acc_ref, t1_ref, es_ref, b1_ref, w2_ref, a2s_ref, a2d_ref,
              t2_ref, adt2_ref, es2_ref):
    acc = acc_ref[...]
    summ = acc[0] + acc[1]
    h = t1_ref[:, 0:D_IN]
    es = es_ref[...]
    u = _head_expand()
    esx = jnp.dot(es, u, preferred_element_type=jnp.float32)
    denx = jnp.dot(summ[:, D_IN:D_IN + HEADS] + es, u,
                   preferred_element_type=jnp.float32)
    num = summ[:, 0:D_IN] + h * esx
    h1 = num / denx + b1_ref[...]
    h1 = jnp.where(h1 > 0, h1, jnp.exp(jnp.minimum(h1, 0.0)) - 1.0)
    h2 = jnp.dot(h1, w2_ref[...], preferred_element_type=jnp.float32)
    a2s = jnp.dot(h2, a2s_ref[...], preferred_element_type=jnp.float32)
    a2d = jnp.dot(h2, a2d_ref[...], preferred_element_type=jnp.float32)
    es2 = jnp.exp(_leaky(a2s + a2d))
    z15 = jnp.zeros((R, 15), jnp.float32)
    t2_ref[...] = jnp.concatenate([h2, a2s, z15], axis=1)
    adt2_ref[...] = jnp.concatenate([a2d, z15], axis=1)
    es2_ref[...] = jnp.broadcast_to(es2, (R, 8))


def _tc3_body(acc_ref, t2_ref, es2_ref, b2_ref, out_ref):
    acc = acc_ref[...]
    summ = acc[0] + acc[1]
    h2 = t2_ref[:, 0:D_OUT]
    es2 = es2_ref[:, 0:1]
    num = summ[:, 0:D_OUT] + h2 * es2
    den = summ[:, D_OUT:D_OUT + 1] + es2
    out_ref[...] = num / (den + 1e-16) + b2_ref[...]


def _tc1(x, w1, asrc, adst, interpret=False):
    grid = N // R
    return pl.pallas_call(
        _tc1_body,
        grid=(grid,),
        in_specs=[
            pl.BlockSpec((R, D_IN), lambda i: (i, 0)),
            pl.BlockSpec((D_IN, D_IN), lambda i: (0, 0)),
            pl.BlockSpec((D_IN, HEADS), lambda i: (0, 0)),
            pl.BlockSpec((D_IN, HEADS), lambda i: (0, 0)),
        ],
        out_specs=[
            pl.BlockSpec((R, TW1), lambda i: (i, 0)),
            pl.BlockSpec((R, 16), lambda i: (i, 0)),
            pl.BlockSpec((R, 8), lambda i: (i, 0)),
        ],
        out_shape=[
            jax.ShapeDtypeStruct((N, TW1), jnp.float32),
            jax.ShapeDtypeStruct((N, 16), jnp.float32),
            jax.ShapeDtypeStruct((N, 8), jnp.float32),
        ],
        interpret=interpret,
    )(x, w1, asrc, adst)


def _tc2(acc1, t1, es1, b1, w2, a2s, a2d, interpret=False):
    grid = N // R
    return pl.pallas_call(
        _tc2_body,
        grid=(grid,),
        in_specs=[
            pl.BlockSpec((NC, R, TW1), lambda i: (0, i, 0)),
            pl.BlockSpec((R, TW1), lambda i: (i, 0)),
            pl.BlockSpec((R, 8), lambda i: (i, 0)),
            pl.BlockSpec((1, D_IN), lambda i: (0, 0)),
            pl.BlockSpec((D_IN, D_OUT), lambda i: (0, 0)),
            pl.BlockSpec((D_OUT, 1), lambda i: (0, 0)),
            pl.BlockSpec((D_OUT, 1), lambda i: (0, 0)),
        ],
        out_specs=[
            pl.BlockSpec((R, TW2), lambda i: (i, 0)),
            pl.BlockSpec((R, 16), lambda i: (i, 0)),
            pl.BlockSpec((R, 8), lambda i: (i, 0)),
        ],
        out_shape=[
            jax.ShapeDtypeStruct((N, TW2), jnp.float32),
            jax.ShapeDtypeStruct((N, 16), jnp.float32),
            jax.ShapeDtypeStruct((N, 8), jnp.float32),
        ],
        interpret=interpret,
    )(acc1, t1, es1, b1, w2, a2s, a2d)


def _tc3(acc2, t2, es2, b2, interpret=False):
    grid = N // R
    return pl.pallas_call(
        _tc3_body,
        grid=(grid,),
        in_specs=[
            pl.BlockSpec((NC, R, TW2), lambda i: (0, i, 0)),
            pl.BlockSpec((R, TW2), lambda i: (i, 0)),
            pl.BlockSpec((R, 8), lambda i: (i, 0)),
            pl.BlockSpec((1, D_OUT), lambda i: (0, 0)),
        ],
        out_specs=pl.BlockSpec((R, D_OUT), lambda i: (i, 0)),
        out_shape=jax.ShapeDtypeStruct((N, D_OUT), jnp.float32),
        interpret=interpret,
    )(acc2, t2, es2, b2)


@jax.jit
def kernel(x, edge_index, W1, att_src1, att_dst1, b1, W2, att_src2,
           att_dst2, b2):
    src = edge_index[0]
    dst = edge_index[1]
    hd_mask = (jnp.arange(D_IN)[:, None] // C1 ==
               jnp.arange(HEADS)[None, :]).astype(jnp.float32)
    asrc1 = hd_mask * att_src1.reshape(D_IN, 1)
    adst1 = hd_mask * att_dst1.reshape(D_IN, 1)
    a2s = att_src2.reshape(D_OUT, 1)
    a2d = att_dst2.reshape(D_OUT, 1)
    b1r = b1.reshape(1, D_IN)
    b2r = b2.reshape(1, D_OUT)

    def make_eip(eb, nchunk):
        pad = nchunk * eb - EPW
        srcp = jnp.pad(src.reshape(NW, EPW), ((0, 0), (0, pad)))
        dstp = jnp.pad(dst.reshape(NW, EPW), ((0, 0), (0, pad)),
                       constant_values=ACC_N - 1)
        return jnp.stack([srcp.reshape(NW, nchunk, eb),
                          dstp.reshape(NW, nchunk, eb)], axis=2)

    eip1 = make_eip(EB1, NCHUNK1)
    eip2 = make_eip(EB2, NCHUNK2)

    t1, adt1, es1 = _tc1(x, W1, asrc1, adst1)
    acc1 = _make_edge_kernel(TW1, D_IN, HEADS, EB1, NCHUNK1)(eip1, t1, adt1)
    t2, adt2, es2 = _tc2(acc1, t1, es1, b1r, W2, a2s, a2d)
    acc2 = _make_edge_kernel(TW2, D_OUT, 1, EB2, NCHUNK2)(eip2, t2, adt2)
    return _tc3(acc2, t2, es2, b2r)


# speedup vs baseline: 1.5248x; 1.5248x over previous
import functools

import jax
import jax.numpy as jnp
from jax import lax
from jax.experimental import pallas as pl
from jax.experimental.pallas import tpu as pltpu
from jax.experimental.pallas import tpu_sc as plsc

N = 10000
E = 320000
D_IN = 128
HEADS = 8
C1 = 16
D_OUT = 64

NC = 2
NS = 16
NW = NC * NS

EPW = E // NW
EB1, NCHUNK1 = 80, 126
EB2, NCHUNK2 = 120, 84

ACC_N = 10240
ROWS_PER_TILE = ACC_N // NS
ZBLK = 80

TW1 = 144
TW2 = 80
R = 1000


def _leaky(t):
    return jnp.maximum(t, 0.2 * t)


@functools.lru_cache(maxsize=None)
def _make_edge_kernel(tw, mw, n_heads, EB, NCHUNK):
    nchunks16 = mw // 16

    mesh = plsc.VectorSubcoreMesh(core_axis_name="c", subcore_axis_name="s",
                                  num_cores=NC, num_subcores=NS)

    NGB, NAB, NIB = 3, 2, 6

    scratch = (
        [pltpu.VMEM_SHARED((ACC_N, tw), jnp.float32)]
        + [pltpu.VMEM((2, EB), jnp.int32) for _ in range(NIB)]
        + [pltpu.VMEM((EB, tw), jnp.float32) for _ in range(NGB)]
        + [pltpu.VMEM((EB, 16), jnp.float32) for _ in range(NAB)]
        + [pltpu.SemaphoreType.DMA for _ in range(NIB + NGB + NAB + NGB)]
    )

    @functools.partial(
        pl.kernel,
        out_type=jax.ShapeDtypeStruct((NC, ACC_N, tw), jnp.float32),
        mesh=mesh,
        scratch_types=scratch,
        compiler_params=pltpu.CompilerParams(use_tc_tiling_on_sc=False),
    )
    def edge_kernel(eip_hbm, table_hbm, adt_hbm, acc_hbm, *scr):
        acc_sh = scr[0]
        ibuf = scr[1:1 + NIB]
        g = scr[1 + NIB:1 + NIB + NGB]
        ad = scr[1 + NIB + NGB:1 + NIB + NGB + NAB]
        sems_all = scr[1 + NIB + NGB + NAB:]
        semi = sems_all[0:NIB]
        semg = sems_all[NIB:NIB + NGB]
        sema = sems_all[NIB + NGB:NIB + NGB + NAB]
        sems = sems_all[NIB + NGB + NAB:]

        c = lax.axis_index("c")
        s = lax.axis_index("s")
        wid = c * NS + s

        def idx_copy(ci, q):
            return pltpu.make_async_copy(eip_hbm.at[wid, ci], ibuf[q], semi[q])

        def g_copy(q, pg):
            return pltpu.make_async_copy(table_hbm.at[ibuf[q].at[0]], g[pg],
                                         semg[pg])

        def ad_copy(q, pa):
            return pltpu.make_async_copy(adt_hbm.at[ibuf[q].at[1]], ad[pa],
                                         sema[pa])

        def sc_copy(q, pg):
            return pltpu.make_async_copy(g[pg], acc_sh.at[ibuf[q].at[1]],
                                         sems[pg])

        zero16 = jnp.zeros((16,), jnp.float32)

        def zrow(r, carry):
            for k in range(tw // 16):
                g[0][r, pl.ds(16 * k, 16)] = zero16
            return carry

        lax.fori_loop(0, ZBLK, zrow, 0)
        row0 = s * ROWS_PER_TILE
        for i in range(ROWS_PER_TILE // ZBLK):
            pltpu.sync_copy(g[0].at[pl.ds(0, ZBLK)],
                            acc_sh.at[pl.ds(row0 + i * ZBLK, ZBLK)])
        plsc.subcore_barrier()

        idx_copy(0, 0).start()
        idx_copy(1, 1).start()
        idx_copy(0, 0).wait()
        g_copy(0, 0).start()
        ad_copy(0, 0).start()

        def group(gi, carry):
            for k in range(6):
                i = gi * 6 + k
                pg = k % 3
                pgn = (k + 1) % 3
                pa = k % 2
                pan = (k + 1) % 2
                qi = k
                qn1 = (k + 1) % 6
                qn2 = (k + 2) % 6

                @pl.when(i >= 2)
                def _():
                    sc_copy((k - 2) % 6, pgn).wait()

                @pl.when(i + 1 < NCHUNK)
                def _():
                    idx_copy(i + 1, qn1).wait()
                    g_copy(qn1, pgn).start()
                    ad_copy(qn1, pan).start()

                @pl.when(i + 2 < NCHUNK)
                def _():
                    idx_copy(i + 2, qn2).start()

                g_copy(qi, pg).wait()
                ad_copy(qi, pa).wait()

                gb = g[pg]
                ab = ad[pa]

                def edge_ea(e, ecarry):
                    t = gb[e, pl.ds(mw, 16)] + ab[e, :]
                    gb[e, pl.ds(mw, 16)] = jnp.exp(jnp.maximum(t, 0.2 * t))
                    return ecarry

                def edge_scale(e, ecarry):
                    ea = gb[e, pl.ds(mw, 16)]
                    for kk in range(nchunks16):
                        lane = (kk * n_heads * 16) // mw
                        gb[e, pl.ds(16 * kk, 16)] = (
                            gb[e, pl.ds(16 * kk, 16)] * ea[lane])
                    return ecarry

                lax.fori_loop(0, EB, edge_ea, 0, unroll=8)
                lax.fori_loop(0, EB, edge_scale, 0, unroll=2)
                pltpu.async_copy(gb, acc_sh.at[ibuf[qi].at[1]], sems[pg],
                                 add=True)
            return carry

        lax.fori_loop(0, NCHUNK // 6, group, 0)

        sc_copy((NCHUNK - 2) % 6, (NCHUNK - 2) % 3).wait()
        sc_copy((NCHUNK - 1) % 6, (NCHUNK - 1) % 3).wait()
        plsc.subcore_barrier()

        for i in range(ROWS_PER_TILE // ZBLK):
            r = row0 + i * ZBLK
            pltpu.sync_copy(acc_sh.at[pl.ds(r, ZBLK)],
                            acc_hbm.at[c, pl.ds(r, ZBLK)])

    return edge_kernel


def _head_expand():
    rows = lax.broadcasted_iota(jnp.int32, (HEADS, D_IN), 0)
    cols = lax.broadcasted_iota(jnp.int32, (HEADS, D_IN), 1)
    return (rows == cols // C1).astype(jnp.float32)


def _tc1_body(x_ref, w1_ref, as_ref, ad_ref, t1_ref, adt_ref, es_ref):
    h = jnp.dot(x_ref[...], w1_ref[...], preferred_element_type=jnp.float32)
    a_s = jnp.dot(h, as_ref[...], preferred_element_type=jnp.float32)
    a_d = jnp.dot(h, ad_ref[...], preferred_element_type=jnp.float32)
    zpad = jnp.zeros((R, 8), jnp.float32)
    t1_ref[...] = jnp.concatenate([h, a_s, zpad], axis=1)
    adt_ref[...] = jnp.concatenate([a_d, zpad], axis=1)
    es_ref[...] = jnp.exp(_leaky(a_s + a_d))


def _tc2_body(acc_ref, t1_ref, es_ref, b1_ref, w2_ref, a2s_ref, a2d_ref,
              t2_ref, adt2_ref, es2_ref):
    acc = acc_ref[...]
    summ = acc[0] + acc[1]
    h = t1_ref[:, 0:D_IN]
    es = es_ref[...]
    u = _head_expand()
    esx = jnp.dot(es, u, preferred_element_type=jnp.float32)
    denx = jnp.dot(summ[:, D_IN:D_IN + HEADS] + es, u,
                   preferred_element_type=jnp.float32)
    num = summ[:, 0:D_IN] + h * esx
    h1 = num / denx + b1_ref[...]
    h1 = jnp.where(h1 > 0, h1, jnp.exp(jnp.minimum(h1, 0.0)) - 1.0)
    h2 = jnp.dot(h1, w2_ref[...], preferred_element_type=jnp.float32)
    a2s = jnp.dot(h2, a2s_ref[...], preferred_element_type=jnp.float32)
    a2d = jnp.dot(h2, a2d_ref[...], preferred_element_type=jnp.float32)
    es2 = jnp.exp(_leaky(a2s + a2d))
    z15 = jnp.zeros((R, 15), jnp.float32)
    t2_ref[...] = jnp.concatenate([h2, a2s, z15], axis=1)
    adt2_ref[...] = jnp.concatenate([a2d, z15], axis=1)
    es2_ref[...] = jnp.broadcast_to(es2, (R, 8))


def _tc3_body(acc_ref, t2_ref, es2_ref, b2_ref, out_ref):
    acc = acc_ref[...]
    summ = acc[0] + acc[1]
    h2 = t2_ref[:, 0:D_OUT]
    es2 = es2_ref[:, 0:1]
    num = summ[:, 0:D_OUT] + h2 * es2
    den = summ[:, D_OUT:D_OUT + 1] + es2
    out_ref[...] = num / (den + 1e-16) + b2_ref[...]


def _tc1(x, w1, asrc, adst, interpret=False):
    grid = N // R
    return pl.pallas_call(
        _tc1_body,
        grid=(grid,),
        in_specs=[
            pl.BlockSpec((R, D_IN), lambda i: (i, 0)),
            pl.BlockSpec((D_IN, D_IN), lambda i: (0, 0)),
            pl.BlockSpec((D_IN, HEADS), lambda i: (0, 0)),
            pl.BlockSpec((D_IN, HEADS), lambda i: (0, 0)),
        ],
        out_specs=[
            pl.BlockSpec((R, TW1), lambda i: (i, 0)),
            pl.BlockSpec((R, 16), lambda i: (i, 0)),
            pl.BlockSpec((R, 8), lambda i: (i, 0)),
        ],
        out_shape=[
            jax.ShapeDtypeStruct((N, TW1), jnp.float32),
            jax.ShapeDtypeStruct((N, 16), jnp.float32),
            jax.ShapeDtypeStruct((N, 8), jnp.float32),
        ],
        interpret=interpret,
    )(x, w1, asrc, adst)


def _tc2(acc1, t1, es1, b1, w2, a2s, a2d, interpret=False):
    grid = N // R
    return pl.pallas_call(
        _tc2_body,
        grid=(grid,),
        in_specs=[
            pl.BlockSpec((NC, R, TW1), lambda i: (0, i, 0)),
            pl.BlockSpec((R, TW1), lambda i: (i, 0)),
            pl.BlockSpec((R, 8), lambda i: (i, 0)),
            pl.BlockSpec((1, D_IN), lambda i: (0, 0)),
            pl.BlockSpec((D_IN, D_OUT), lambda i: (0, 0)),
            pl.BlockSpec((D_OUT, 1), lambda i: (0, 0)),
            pl.BlockSpec((D_OUT, 1), lambda i: (0, 0)),
        ],
        out_specs=[
            pl.BlockSpec((R, TW2), lambda i: (i, 0)),
            pl.BlockSpec((R, 16), lambda i: (i, 0)),
            pl.BlockSpec((R, 8), lambda i: (i, 0)),
        ],
        out_shape=[
            jax.ShapeDtypeStruct((N, TW2), jnp.float32),
            jax.ShapeDtypeStruct((N, 16), jnp.float32),
            jax.ShapeDtypeStruct((N, 8), jnp.float32),
        ],
        interpret=interpret,
    )(acc1, t1, es1, b1, w2, a2s, a2d)


def _tc3(acc2, t2, es2, b2, interpret=False):
    grid = N // R
    return pl.pallas_call(
        _tc3_body,
        grid=(grid,),
        in_specs=[
            pl.BlockSpec((NC, R, TW2), lambda i: (0, i, 0)),
            pl.BlockSpec((R, TW2), lambda i: (i, 0)),
            pl.BlockSpec((R, 8), lambda i: (i, 0)),
            pl.BlockSpec((1, D_OUT), lambda i: (0, 0)),
        ],
        out_specs=pl.BlockSpec((R, D_OUT), lambda i: (i, 0)),
        out_shape=jax.ShapeDtypeStruct((N, D_OUT), jnp.float32),
        interpret=interpret,
    )(acc2, t2, es2, b2)


@jax.jit
def kernel(x, edge_index, W1, att_src1, att_dst1, b1, W2, att_src2,
           att_dst2, b2):
    src = edge_index[0]
    dst = edge_index[1]
    hd_mask = (jnp.arange(D_IN)[:, None] // C1 ==
               jnp.arange(HEADS)[None, :]).astype(jnp.float32)
    asrc1 = hd_mask * att_src1.reshape(D_IN, 1)
    adst1 = hd_mask * att_dst1.reshape(D_IN, 1)
    a2s = att_src2.reshape(D_OUT, 1)
    a2d = att_dst2.reshape(D_OUT, 1)
    b1r = b1.reshape(1, D_IN)
    b2r = b2.reshape(1, D_OUT)

    def make_eip(eb, nchunk):
        pad = nchunk * eb - EPW
        srcp = jnp.pad(src.reshape(NW, EPW), ((0, 0), (0, pad)))
        dstp = jnp.pad(dst.reshape(NW, EPW), ((0, 0), (0, pad)),
                       constant_values=ACC_N - 1)
        return jnp.stack([srcp.reshape(NW, nchunk, eb),
                          dstp.reshape(NW, nchunk, eb)], axis=2)

    eip1 = make_eip(EB1, NCHUNK1)
    eip2 = make_eip(EB2, NCHUNK2)

    t1, adt1, es1 = _tc1(x, W1, asrc1, adst1)
    acc1 = _make_edge_kernel(TW1, D_IN, HEADS, EB1, NCHUNK1)(eip1, t1, adt1)
    t2, adt2, es2 = _tc2(acc1, t1, es1, b1r, W2, a2s, a2d)
    acc2 = _make_edge_kernel(TW2, D_OUT, 1, EB2, NCHUNK2)(eip2, t2, adt2)
    return _tc3(acc2, t2, es2, b2r)


# speedup vs baseline: 1.9838x; 1.3010x over previous
import functools

import jax
import jax.numpy as jnp
from jax import lax
from jax.experimental import pallas as pl
from jax.experimental.pallas import tpu as pltpu
from jax.experimental.pallas import tpu_sc as plsc

N = 10000
E = 320000
D_IN = 128
HEADS = 8
C1 = 16
D_OUT = 64

NC = 2
NS = 16
NW = NC * NS

EPW = E // NW
EB1, NCHUNK1 = 80, 126
EB2, NCHUNK2 = 120, 84

ACC_N = 10240
ROWS_PER_TILE = ACC_N // NS
ZBLK = 80

TW1 = 144
TW2 = 80
R = 1000


def _leaky(t):
    return jnp.maximum(t, 0.2 * t)


@functools.lru_cache(maxsize=None)
def _make_edge_kernel(tw, mw, n_heads, EB, NCHUNK):
    nchunks16 = mw // 16

    mesh = plsc.VectorSubcoreMesh(core_axis_name="c", subcore_axis_name="s",
                                  num_cores=NC, num_subcores=NS)

    NGB, NAB, NIB = 3, 2, 6

    scratch = (
        [pltpu.VMEM_SHARED((ACC_N, tw), jnp.float32)]
        + [pltpu.VMEM((2, EB), jnp.int32) for _ in range(NIB)]
        + [pltpu.VMEM((EB, tw), jnp.float32) for _ in range(NGB)]
        + [pltpu.VMEM((EB, 16), jnp.float32) for _ in range(NAB)]
        + [pltpu.SemaphoreType.DMA for _ in range(NIB + NGB + NAB + NGB)]
    )

    @functools.partial(
        pl.kernel,
        out_type=jax.ShapeDtypeStruct((NC, ACC_N, tw), jnp.float32),
        mesh=mesh,
        scratch_types=scratch,
        compiler_params=pltpu.CompilerParams(use_tc_tiling_on_sc=False),
    )
    def edge_kernel(eip_hbm, table_hbm, adt_hbm, acc_hbm, *scr):
        acc_sh = scr[0]
        ibuf = scr[1:1 + NIB]
        g = scr[1 + NIB:1 + NIB + NGB]
        ad = scr[1 + NIB + NGB:1 + NIB + NGB + NAB]
        sems_all = scr[1 + NIB + NGB + NAB:]
        semi = sems_all[0:NIB]
        semg = sems_all[NIB:NIB + NGB]
        sema = sems_all[NIB + NGB:NIB + NGB + NAB]
        sems = sems_all[NIB + NGB + NAB:]

        c = lax.axis_index("c")
        s = lax.axis_index("s")
        wid = c * NS + s

        def idx_copy(ci, q):
            return pltpu.make_async_copy(eip_hbm.at[wid, ci], ibuf[q], semi[q])

        def g_copy(q, pg):
            return pltpu.make_async_copy(table_hbm.at[ibuf[q].at[0]], g[pg],
                                         semg[pg])

        def ad_copy(q, pa):
            return pltpu.make_async_copy(adt_hbm.at[ibuf[q].at[1]], ad[pa],
                                         sema[pa])

        def sc_copy(q, pg):
            return pltpu.make_async_copy(g[pg], acc_sh.at[ibuf[q].at[1]],
                                         sems[pg])

        zero16 = jnp.zeros((16,), jnp.float32)

        def zrow(r, carry):
            for k in range(tw // 16):
                g[0][r, pl.ds(16 * k, 16)] = zero16
            return carry

        lax.fori_loop(0, ZBLK, zrow, 0)
        row0 = s * ROWS_PER_TILE
        for i in range(ROWS_PER_TILE // ZBLK):
            pltpu.sync_copy(g[0].at[pl.ds(0, ZBLK)],
                            acc_sh.at[pl.ds(row0 + i * ZBLK, ZBLK)])
        plsc.subcore_barrier()

        idx_copy(0, 0).start()
        idx_copy(1, 1).start()
        idx_copy(0, 0).wait()
        g_copy(0, 0).start()
        ad_copy(0, 0).start()

        def group(gi, carry):
            for k in range(6):
                i = gi * 6 + k
                pg = k % 3
                pgn = (k + 1) % 3
                pa = k % 2
                pan = (k + 1) % 2
                qi = k
                qn1 = (k + 1) % 6
                qn2 = (k + 2) % 6

                @pl.when(i >= 2)
                def _():
                    sc_copy((k - 2) % 6, pgn).wait()

                @pl.when(i + 1 < NCHUNK)
                def _():
                    idx_copy(i + 1, qn1).wait()
                    g_copy(qn1, pgn).start()
                    ad_copy(qn1, pan).start()

                @pl.when(i + 2 < NCHUNK)
                def _():
                    idx_copy(i + 2, qn2).start()

                g_copy(qi, pg).wait()
                ad_copy(qi, pa).wait()

                gb = g[pg]
                ab = ad[pa]

                @plsc.parallel_loop(0, EB, unroll=4)
                def _(e):
                    t = gb[e, pl.ds(mw, 16)] + ab[e, :]
                    ea = jnp.exp(jnp.maximum(t, 0.2 * t))
                    gb[e, pl.ds(mw, 16)] = ea
                    for kk in range(nchunks16):
                        lane = (kk * n_heads * 16) // mw
                        gb[e, pl.ds(16 * kk, 16)] = (
                            gb[e, pl.ds(16 * kk, 16)] * ea[lane])
                pltpu.async_copy(gb, acc_sh.at[ibuf[qi].at[1]], sems[pg],
                                 add=True)
            return carry

        lax.fori_loop(0, NCHUNK // 6, group, 0)

        sc_copy((NCHUNK - 2) % 6, (NCHUNK - 2) % 3).wait()
        sc_copy((NCHUNK - 1) % 6, (NCHUNK - 1) % 3).wait()
        plsc.subcore_barrier()

        for i in range(ROWS_PER_TILE // ZBLK):
            r = row0 + i * ZBLK
            pltpu.sync_copy(acc_sh.at[pl.ds(r, ZBLK)],
                            acc_hbm.at[c, pl.ds(r, ZBLK)])

    return edge_kernel


def _head_expand():
    rows = lax.broadcasted_iota(jnp.int32, (HEADS, D_IN), 0)
    cols = lax.broadcasted_iota(jnp.int32, (HEADS, D_IN), 1)
    return (rows == cols // C1).astype(jnp.float32)


def _tc1_body(x_ref, w1_ref, as_ref, ad_ref, t1_ref, adt_ref, es_ref):
    h = jnp.dot(x_ref[...], w1_ref[...], preferred_element_type=jnp.float32)
    a_s = jnp.dot(h, as_ref[...], preferred_element_type=jnp.float32)
    a_d = jnp.dot(h, ad_ref[...], preferred_element_type=jnp.float32)
    zpad = jnp.zeros((R, 8), jnp.float32)
    t1_ref[...] = jnp.concatenate([h, a_s, zpad], axis=1)
    adt_ref[...] = jnp.concatenate([a_d, zpad], axis=1)
    es_ref[...] = jnp.exp(_leaky(a_s + a_d))


def _tc2_body(acc_ref, t1_ref, es_ref, b1_ref, w2_ref, a2s_ref, a2d_ref,
              t2_ref, adt2_ref, es2_ref):
    acc = acc_ref[...]
    summ = acc[0] + acc[1]
    h = t1_ref[:, 0:D_IN]
    es = es_ref[...]
    u = _head_expand()
    esx = jnp.dot(es, u, preferred_element_type=jnp.float32)
    denx = jnp.dot(summ[:, D_IN:D_IN + HEADS] + es, u,
                   preferred_element_type=jnp.float32)
    num = summ[:, 0:D_IN] + h * esx
    h1 = num / denx + b1_ref[...]
    h1 = jnp.where(h1 > 0, h1, jnp.exp(jnp.minimum(h1, 0.0)) - 1.0)
    h2 = jnp.dot(h1, w2_ref[...], preferred_element_type=jnp.float32)
    a2s = jnp.dot(h2, a2s_ref[...], preferred_element_type=jnp.float32)
    a2d = jnp.dot(h2, a2d_ref[...], preferred_element_type=jnp.float32)
    es2 = jnp.exp(_leaky(a2s + a2d))
    z15 = jnp.zeros((R, 15), jnp.float32)
    t2_ref[...] = jnp.concatenate([h2, a2s, z15], axis=1)
    adt2_ref[...] = jnp.concatenate([a2d, z15], axis=1)
    es2_ref[...] = jnp.broadcast_to(es2, (R, 8))


def _tc3_body(acc_ref, t2_ref, es2_ref, b2_ref, out_ref):
    acc = acc_ref[...]
    summ = acc[0] + acc[1]
    h2 = t2_ref[:, 0:D_OUT]
    es2 = es2_ref[:, 0:1]
    num = summ[:, 0:D_OUT] + h2 * es2
    den = summ[:, D_OUT:D_OUT + 1] + es2
    out_ref[...] = num / (den + 1e-16) + b2_ref[...]


def _tc1(x, w1, asrc, adst, interpret=False):
    grid = N // R
    return pl.pallas_call(
        _tc1_body,
        grid=(grid,),
        in_specs=[
            pl.BlockSpec((R, D_IN), lambda i: (i, 0)),
            pl.BlockSpec((D_IN, D_IN), lambda i: (0, 0)),
            pl.BlockSpec((D_IN, HEADS), lambda i: (0, 0)),
            pl.BlockSpec((D_IN, HEADS), lambda i: (0, 0)),
        ],
        out_specs=[
            pl.BlockSpec((R, TW1), lambda i: (i, 0)),
            pl.BlockSpec((R, 16), lambda i: (i, 0)),
            pl.BlockSpec((R, 8), lambda i: (i, 0)),
        ],
        out_shape=[
            jax.ShapeDtypeStruct((N, TW1), jnp.float32),
            jax.ShapeDtypeStruct((N, 16), jnp.float32),
            jax.ShapeDtypeStruct((N, 8), jnp.float32),
        ],
        interpret=interpret,
    )(x, w1, asrc, adst)


def _tc2(acc1, t1, es1, b1, w2, a2s, a2d, interpret=False):
    grid = N // R
    return pl.pallas_call(
        _tc2_body,
        grid=(grid,),
        in_specs=[
            pl.BlockSpec((NC, R, TW1), lambda i: (0, i, 0)),
            pl.BlockSpec((R, TW1), lambda i: (i, 0)),
            pl.BlockSpec((R, 8), lambda i: (i, 0)),
            pl.BlockSpec((1, D_IN), lambda i: (0, 0)),
            pl.BlockSpec((D_IN, D_OUT), lambda i: (0, 0)),
            pl.BlockSpec((D_OUT, 1), lambda i: (0, 0)),
            pl.BlockSpec((D_OUT, 1), lambda i: (0, 0)),
        ],
        out_specs=[
            pl.BlockSpec((R, TW2), lambda i: (i, 0)),
            pl.BlockSpec((R, 16), lambda i: (i, 0)),
            pl.BlockSpec((R, 8), lambda i: (i, 0)),
        ],
        out_shape=[
            jax.ShapeDtypeStruct((N, TW2), jnp.float32),
            jax.ShapeDtypeStruct((N, 16), jnp.float32),
            jax.ShapeDtypeStruct((N, 8), jnp.float32),
        ],
        interpret=interpret,
    )(acc1, t1, es1, b1, w2, a2s, a2d)


def _tc3(acc2, t2, es2, b2, interpret=False):
    grid = N // R
    return pl.pallas_call(
        _tc3_body,
        grid=(grid,),
        in_specs=[
            pl.BlockSpec((NC, R, TW2), lambda i: (0, i, 0)),
            pl.BlockSpec((R, TW2), lambda i: (i, 0)),
            pl.BlockSpec((R, 8), lambda i: (i, 0)),
            pl.BlockSpec((1, D_OUT), lambda i: (0, 0)),
        ],
        out_specs=pl.BlockSpec((R, D_OUT), lambda i: (i, 0)),
        out_shape=jax.ShapeDtypeStruct((N, D_OUT), jnp.float32),
        interpret=interpret,
    )(acc2, t2, es2, b2)


@jax.jit
def kernel(x, edge_index, W1, att_src1, att_dst1, b1, W2, att_src2,
           att_dst2, b2):
    src = edge_index[0]
    dst = edge_index[1]
    hd_mask = (jnp.arange(D_IN)[:, None] // C1 ==
               jnp.arange(HEADS)[None, :]).astype(jnp.float32)
    asrc1 = hd_mask * att_src1.reshape(D_IN, 1)
    adst1 = hd_mask * att_dst1.reshape(D_IN, 1)
    a2s = att_src2.reshape(D_OUT, 1)
    a2d = att_dst2.reshape(D_OUT, 1)
    b1r = b1.reshape(1, D_IN)
    b2r = b2.reshape(1, D_OUT)

    def make_eip(eb, nchunk):
        pad = nchunk * eb - EPW
        srcp = jnp.pad(src.reshape(NW, EPW), ((0, 0), (0, pad)))
        dstp = jnp.pad(dst.reshape(NW, EPW), ((0, 0), (0, pad)),
                       constant_values=ACC_N - 1)
        return jnp.stack([srcp.reshape(NW, nchunk, eb),
                          dstp.reshape(NW, nchunk, eb)], axis=2)

    eip1 = make_eip(EB1, NCHUNK1)
    eip2 = make_eip(EB2, NCHUNK2)

    t1, adt1, es1 = _tc1(x, W1, asrc1, adst1)
    acc1 = _make_edge_kernel(TW1, D_IN, HEADS, EB1, NCHUNK1)(eip1, t1, adt1)
    t2, adt2, es2 = _tc2(acc1, t1, es1, b1r, W2, a2s, a2d)
    acc2 = _make_edge_kernel(TW2, D_OUT, 1, EB2, NCHUNK2)(eip2, t2, adt2)
    return _tc3(acc2, t2, es2, b2r)
